# baseline (speedup 1.0000x reference)
import functools

import jax
import jax.numpy as jnp
from jax import lax
from jax.experimental import pallas as pl
from jax.experimental.pallas import tpu as pltpu
from jax.experimental.pallas import tpu_sc as plsc

_CW = 128
_NTOK = 1000000
_NFULL = _NTOK // _CW
_NCH = _NFULL + 1
_TAILW = _NTOK - _NFULL * _CW
_NR = 12288
_CAP = 128
_NROUT = _NR + _CAP


def _make_sc_stream_gather(d: int):
    info = plsc.get_sparse_core_info()
    nc, ns, nl = info.num_cores, info.num_subcores, info.num_lanes
    nw = nc * ns
    base_per_w = _NCH // nw
    extra = _NCH % nw
    max_chunks = base_per_w + 1

    mesh = plsc.VectorSubcoreMesh(core_axis_name="c", subcore_axis_name="s")

    @functools.partial(
        pl.kernel,
        mesh=mesh,
        out_type=jax.ShapeDtypeStruct((_NROUT, 128), jnp.float32),
        scratch_types=[
            pltpu.VMEM((_NR + 16,), jnp.int32),
            pltpu.VMEM((_NR + 16,), jnp.int32),
            pltpu.VMEM((d, _CW), jnp.float32),
            pltpu.VMEM((d, _CW), jnp.float32),
            pltpu.VMEM((d, _CW), jnp.float32),
            pltpu.VMEM((d, _CW), jnp.float32),
            pltpu.VMEM((d, _CW), jnp.float32),
            pltpu.VMEM((d, _CW), jnp.float32),
            pltpu.VMEM((d, _CW), jnp.float32),
            pltpu.VMEM((d, _CW), jnp.float32),
            pltpu.VMEM((_CAP, 128), jnp.float32),
            pltpu.VMEM((_CAP,), jnp.int32),
            pltpu.SMEM((8,), jnp.int32),
            pltpu.SemaphoreType.DMA,
            pltpu.SemaphoreType.DMA,
            pltpu.SemaphoreType.DMA,
            pltpu.SemaphoreType.DMA,
            pltpu.SemaphoreType.DMA,
            pltpu.SemaphoreType.DMA,
            pltpu.SemaphoreType.DMA,
            pltpu.SemaphoreType.DMA,
            pltpu.SemaphoreType.DMA,
        ],
        compiler_params=pltpu.CompilerParams(needs_layout_passes=False),
    )
    def gather_kernel(wt_hbm, wtail_hbm, sids_hbm, pos_hbm, out_hbm,
                      ids_v, pos_v, buf0, buf1, buf2, buf3, buf4, buf5, buf6,
                      buf7, rows_v, posb_v, pc_s, sem0, sem1, sem2, sem3,
                      sem4, sem5, sem6, sem7, semo):
        wid = lax.axis_index("s") * nc + lax.axis_index("c")
        kw0 = base_per_w * wid + jnp.minimum(wid, extra)
        kw1 = kw0 + base_per_w + jnp.where(wid < extra, 1, 0)
        pltpu.sync_copy(sids_hbm, ids_v.at[pl.ds(0, _NR)])
        pltpu.sync_copy(pos_hbm, pos_v.at[pl.ds(0, _NR)])
        lane = lax.iota(jnp.int32, nl)

        def reinit_posb():
            @pl.loop(0, _CAP // nl)
            def _ri(i):
                posb_v[pl.ds(i * nl, nl)] = _NR + i * nl + lane

        def flush():
            pltpu.async_copy(rows_v, out_hbm.at[posb_v], semo).wait()
            reinit_posb()

        reinit_posb()

        def val_at(j):
            jj = (j // nl) * nl
            vec = ids_v[pl.ds(jj, nl)]
            return jnp.sum(jnp.where(lane == j - jj, vec, 0))

        v0 = kw0 * _CW
        p = jnp.int32(0)
        for s in range(13, -1, -1):
            cand = p + (1 << s)
            take = jnp.logical_and(cand <= _NR, val_at(cand - 1) < v0)
            p = jnp.where(take, cand, p)
        pc_s[0] = 0
        pc_s[1] = p

        def start_dma(k, buf, sem):
            c0 = pl.multiple_of(k * _CW, _CW)
            is_tail = k == _NFULL
            @pl.when(jnp.logical_not(is_tail))
            def _():
                pltpu.async_copy(wt_hbm.at[:, pl.ds(c0, _CW)], buf, sem)
            @pl.when(is_tail)
            def _():
                pltpu.async_copy(wtail_hbm, buf.at[:, pl.ds(0, 128)], sem)

        def wait_dma(k, buf, sem):
            is_tail = k == _NFULL
            @pl.when(jnp.logical_not(is_tail))
            def _():
                pltpu.make_async_copy(wt_hbm.at[:, pl.ds(0, _CW)], buf,
                                      sem).wait()
            @pl.when(is_tail)
            def _():
                pltpu.make_async_copy(
                    wtail_hbm, buf.at[:, pl.ds(0, 128)], sem).wait()

        def process(k, buf):
            c0 = k * _CW
            cend = c0 + _CW

            def cond(st):
                return jnp.logical_and(st[1] == 0, st[0] < _NR)

            def body(st):
                pp = st[0]
                base = (pp // nl) * nl
                ids16 = ids_v[pl.ds(base, nl)]
                pos16 = pos_v[pl.ds(base, nl)]
                t = base + lane
                mask = jnp.logical_and(t >= pp, ids16 < cend)
                msel = mask.astype(jnp.int32)
                nsel = jnp.sum(msel)
                cnt = pc_s[0]
                dst16 = jnp.clip(cnt + plsc.cumsum(msel) - 1, 0, _CAP - 1)
                colsel = jnp.clip(ids16 - c0, 0, _CW - 1)
                plsc.store_scatter(posb_v, [dst16], pos16, mask=mask)

                @pl.loop(0, d)
                def _col(cc):
                    csplat = jnp.zeros((nl,), jnp.int32) + cc
                    vals = plsc.load_gather(buf, [csplat, colsel], mask=mask)
                    plsc.store_scatter(rows_v, [dst16, csplat], vals,
                                       mask=mask)

                cnt2 = cnt + nsel
                pc_s[0] = cnt2

                @pl.when(cnt2 > _CAP - nl)
                def _():
                    flush()
                    pc_s[0] = 0

                over = jnp.sum(jnp.where(ids16 >= cend, 1, 0))
                return (pp + nsel, jnp.where(over > 0, 1, 0))

            pf = lax.while_loop(cond, body, (pc_s[1], jnp.int32(0)))
            pc_s[1] = pf[0]

        ring = [(buf0, sem0), (buf1, sem1), (buf2, sem2), (buf3, sem3),
                (buf4, sem4), (buf5, sem5), (buf6, sem6), (buf7, sem7)]
        nring = len(ring)
        for off, (buf, sem) in enumerate(ring):
            @pl.when(kw0 + off < kw1)
            def _(off=off, buf=buf, sem=sem):
                start_dma(kw0 + off, buf, sem)

        @pl.loop(0, (max_chunks + nring - 1) // nring)
        def _round(ti):
            k = kw0 + nring * ti
            for off, (buf, sem) in enumerate(ring):
                ko = k + off

                @pl.when(ko < kw1)
                def _(ko=ko, buf=buf, sem=sem):
                    wait_dma(ko, buf, sem)
                    process(ko, buf)

                    @pl.when(ko + nring < kw1)
                    def _(ko=ko, buf=buf, sem=sem):
                        start_dma(ko + nring, buf, sem)

        flush()

    return gather_kernel


def _loss_body(xt_ref, sw_ref, tw_ref, o_ref):
    xt = xt_ref[...]
    sw = sw_ref[...][:, :64]
    tw = tw_ref[...][:, :64]
    logits = lax.dot_general(
        xt, sw, (((0,), (1,)), ((), ())),
        preferred_element_type=jnp.float32,
    )
    s = jnp.sum(jnp.exp(logits), axis=1, keepdims=True)
    t = jnp.sum(xt.T * tw, axis=1, keepdims=True)
    o_ref[...] = jnp.log(s) - t


def kernel(inputs, labels, sample_ids, weight):
    batch, d = inputs.shape
    nsamp = sample_ids.shape[0]

    ids = jnp.concatenate([sample_ids, labels])
    pos = jnp.arange(_NR, dtype=jnp.int32)
    sorted_ids, sorted_pos = lax.sort_key_val(ids, pos)

    wt = weight.T
    wtail = jnp.pad(
        lax.slice(weight, (_NFULL * _CW, 0), (_NTOK, d)).T,
        ((0, 0), (0, 128 - _TAILW)),
    )
    gathered = _make_sc_stream_gather(d)(wt, wtail, sorted_ids, sorted_pos)

    bm = 256
    out = pl.pallas_call(
        _loss_body,
        grid=(batch // bm,),
        in_specs=[
            pl.BlockSpec((d, bm), lambda i: (0, i)),
            pl.BlockSpec((nsamp, 128), lambda i: (0, 0)),
            pl.BlockSpec((bm, 128), lambda i: (i + nsamp // bm, 0)),
        ],
        out_specs=pl.BlockSpec((bm, 1), lambda i: (i, 0)),
        out_shape=jax.ShapeDtypeStruct((batch, 1), jnp.float32),
    )(inputs.T, gathered, gathered)
    return out[:, 0]


# speedup vs baseline: 1.0519x; 1.0519x over previous
import functools

import jax
import jax.numpy as jnp
from jax import lax
from jax.experimental import pallas as pl
from jax.experimental.pallas import tpu as pltpu
from jax.experimental.pallas import tpu_sc as plsc

_CW = 256
_NTOK = 1000000
_NFULL = _NTOK // _CW
_NCH = _NFULL + 1
_TAILW = _NTOK - _NFULL * _CW
_NR = 12288
_CAP = 160
_NROUT = _NR + _CAP


def _make_sc_stream_gather(d: int):
    info = plsc.get_sparse_core_info()
    nc, ns, nl = info.num_cores, info.num_subcores, info.num_lanes
    nw = nc * ns
    base_per_w = _NCH // nw
    extra = _NCH % nw
    max_chunks = base_per_w + 1

    mesh = plsc.VectorSubcoreMesh(core_axis_name="c", subcore_axis_name="s")

    @functools.partial(
        pl.kernel,
        mesh=mesh,
        out_type=jax.ShapeDtypeStruct((_NROUT, 128), jnp.float32),
        scratch_types=[
            pltpu.VMEM((_NR + 16,), jnp.int32),
            pltpu.VMEM((_NR + 16,), jnp.int32),
            pltpu.VMEM((d, _CW), jnp.float32),
            pltpu.VMEM((d, _CW), jnp.float32),
            pltpu.VMEM((d, _CW), jnp.float32),
            pltpu.VMEM((d, _CW), jnp.float32),
            pltpu.VMEM((d, _CW), jnp.float32),
            pltpu.VMEM((_CAP, 128), jnp.float32),
            pltpu.VMEM((_CAP,), jnp.int32),
            pltpu.SMEM((8,), jnp.int32),
            pltpu.SemaphoreType.DMA,
            pltpu.SemaphoreType.DMA,
            pltpu.SemaphoreType.DMA,
            pltpu.SemaphoreType.DMA,
            pltpu.SemaphoreType.DMA,
            pltpu.SemaphoreType.DMA,
        ],
        compiler_params=pltpu.CompilerParams(needs_layout_passes=False),
    )
    def gather_kernel(wt_hbm, wtail_hbm, sids_hbm, pos_hbm, out_hbm,
                      ids_v, pos_v, buf0, buf1, buf2, buf3, buf4, rows_v,
                      posb_v, pc_s, sem0, sem1, sem2, sem3, sem4, semo):
        wid = lax.axis_index("s") * nc + lax.axis_index("c")
        kw0 = base_per_w * wid + jnp.minimum(wid, extra)
        kw1 = kw0 + base_per_w + jnp.where(wid < extra, 1, 0)
        pltpu.sync_copy(sids_hbm, ids_v.at[pl.ds(0, _NR)])
        pltpu.sync_copy(pos_hbm, pos_v.at[pl.ds(0, _NR)])
        lane = lax.iota(jnp.int32, nl)

        def reinit_posb():
            @pl.loop(0, _CAP // nl)
            def _ri(i):
                posb_v[pl.ds(i * nl, nl)] = _NR + i * nl + lane

        def flush():
            pltpu.async_copy(rows_v, out_hbm.at[posb_v], semo).wait()
            reinit_posb()

        reinit_posb()

        def val_at(j):
            jj = (j // nl) * nl
            vec = ids_v[pl.ds(jj, nl)]
            return jnp.sum(jnp.where(lane == j - jj, vec, 0))

        v0 = kw0 * _CW
        p = jnp.int32(0)
        for s in range(13, -1, -1):
            cand = p + (1 << s)
            take = jnp.logical_and(cand <= _NR, val_at(cand - 1) < v0)
            p = jnp.where(take, cand, p)
        pc_s[0] = 0
        pc_s[1] = p

        def start_dma(k, buf, sem):
            c0 = pl.multiple_of(k * _CW, _CW)
            is_tail = k == _NFULL
            @pl.when(jnp.logical_not(is_tail))
            def _():
                pltpu.async_copy(wt_hbm.at[:, pl.ds(c0, _CW)], buf, sem)
            @pl.when(is_tail)
            def _():
                pltpu.async_copy(wtail_hbm, buf.at[:, pl.ds(0, 128)], sem)

        def wait_dma(k, buf, sem):
            is_tail = k == _NFULL
            @pl.when(jnp.logical_not(is_tail))
            def _():
                pltpu.make_async_copy(wt_hbm.at[:, pl.ds(0, _CW)], buf,
                                      sem).wait()
            @pl.when(is_tail)
            def _():
                pltpu.make_async_copy(
                    wtail_hbm, buf.at[:, pl.ds(0, 128)], sem).wait()

        def process(k, buf):
            c0 = k * _CW
            cend = c0 + _CW

            def cond(st):
                return jnp.logical_and(st[1] == 0, st[0] < _NR)

            def body(st):
                pp = st[0]
                base = (pp // nl) * nl
                ids16 = ids_v[pl.ds(base, nl)]
                pos16 = pos_v[pl.ds(base, nl)]
                t = base + lane
                mask = jnp.logical_and(t >= pp, ids16 < cend)
                msel = mask.astype(jnp.int32)
                nsel = jnp.sum(msel)
                cnt = pc_s[0]
                dst16 = jnp.clip(cnt + plsc.cumsum(msel) - 1, 0, _CAP - 1)
                colsel = jnp.clip(ids16 - c0, 0, _CW - 1)
                plsc.store_scatter(posb_v, [dst16], pos16, mask=mask)

                @pl.loop(0, d)
                def _col(cc):
                    csplat = jnp.zeros((nl,), jnp.int32) + cc
                    vals = plsc.load_gather(buf, [csplat, colsel], mask=mask)
                    plsc.store_scatter(rows_v, [dst16, csplat], vals,
                                       mask=mask)

                cnt2 = cnt + nsel
                pc_s[0] = cnt2

                @pl.when(cnt2 > _CAP - nl)
                def _():
                    flush()
                    pc_s[0] = 0

                over = jnp.sum(jnp.where(ids16 >= cend, 1, 0))
                return (pp + nsel, jnp.where(over > 0, 1, 0))

            pf = lax.while_loop(cond, body, (pc_s[1], jnp.int32(0)))
            pc_s[1] = pf[0]

        ring = [(buf0, sem0), (buf1, sem1), (buf2, sem2), (buf3, sem3),
                (buf4, sem4)]
        nring = len(ring)
        for off, (buf, sem) in enumerate(ring):
            @pl.when(kw0 + off < kw1)
            def _(off=off, buf=buf, sem=sem):
                start_dma(kw0 + off, buf, sem)

        @pl.loop(0, (max_chunks + nring - 1) // nring)
        def _round(ti):
            k = kw0 + nring * ti
            for off, (buf, sem) in enumerate(ring):
                ko = k + off

                @pl.when(ko < kw1)
                def _(ko=ko, buf=buf, sem=sem):
                    wait_dma(ko, buf, sem)
                    process(ko, buf)

                    @pl.when(ko + nring < kw1)
                    def _(ko=ko, buf=buf, sem=sem):
                        start_dma(ko + nring, buf, sem)

        flush()

    return gather_kernel


def _loss_body(xt_ref, sw_ref, tw_ref, o_ref):
    xt = xt_ref[...]
    sw = sw_ref[...][:, :64]
    tw = tw_ref[...][:, :64]
    logits = lax.dot_general(
        xt, sw, (((0,), (1,)), ((), ())),
        preferred_element_type=jnp.float32,
    )
    s = jnp.sum(jnp.exp(logits), axis=1, keepdims=True)
    t = jnp.sum(xt.T * tw, axis=1, keepdims=True)
    o_ref[...] = jnp.log(s) - t


def kernel(inputs, labels, sample_ids, weight):
    batch, d = inputs.shape
    nsamp = sample_ids.shape[0]

    ids = jnp.concatenate([sample_ids, labels])
    pos = jnp.arange(_NR, dtype=jnp.int32)
    sorted_ids, sorted_pos = lax.sort_key_val(ids, pos)

    wt = weight.T
    wtail = jnp.pad(
        lax.slice(weight, (_NFULL * _CW, 0), (_NTOK, d)).T,
        ((0, 0), (0, 128 - _TAILW)),
    )
    gathered = _make_sc_stream_gather(d)(wt, wtail, sorted_ids, sorted_pos)

    bm = 256
    out = pl.pallas_call(
        _loss_body,
        grid=(batch // bm,),
        in_specs=[
            pl.BlockSpec((d, bm), lambda i: (0, i)),
            pl.BlockSpec((nsamp, 128), lambda i: (0, 0)),
            pl.BlockSpec((bm, 128), lambda i: (i + nsamp // bm, 0)),
        ],
        out_specs=pl.BlockSpec((bm, 1), lambda i: (i, 0)),
        out_shape=jax.ShapeDtypeStruct((batch, 1), jnp.float32),
    )(inputs.T, gathered, gathered)
    return out[:, 0]
